# baseline (speedup 1.0000x reference)
import functools

import jax
import jax.numpy as jnp
from jax import lax
from jax.experimental import pallas as pl
from jax.experimental.pallas import tpu as pltpu
from jax.experimental.pallas import tpu_sc as plsc

N = 10000
E = 160000
DIN = 128
DE = 16
H = 128
HH = 64
OUT = 8
MAX_ACTION = 1.0

NCORE = 2
NSUB = 16
PER_TILE = E // NSUB
K = 40
NCHUNK = PER_TILE // K
NPAD = 10112
NPT = NPAD // NSUB

_f32 = jnp.float32
_RCP_MAGIC = 0x7EF127EA


def _sigmoid(en):
    w = 1.0 + jnp.exp(-jnp.maximum(en, -60.0))
    y = plsc.bitcast(_RCP_MAGIC - plsc.bitcast(w, jnp.int32), _f32)
    y = y * (2.0 - w * y)
    y = y * (2.0 - w * y)
    return y


def _dot(a, b):
    return jax.lax.dot_general(a, b, (((1,), (0,)), ((), ())),
                               preferred_element_type=_f32)


def _prep_body(x_ref, ehw_ref, ehb_ref, aw_ref, ab_ref, bw_ref, bb_ref,
               dw_ref, db_ref, ew_ref, eb_ref, eew_ref, eeb_ref,
               cw0_ref, cb0_ref, cw1_ref, cb1_ref,
               h_out, ah_out, dbt_out, et_out, we0_out, be0_out,
               we1_out, be1_out):
    h = _dot(x_ref[...], ehw_ref[...]) + ehb_ref[...]
    h_out[...] = h
    ah_out[...] = _dot(h, aw_ref[...]) + ab_ref[...]
    bh = _dot(h, bw_ref[...]) + bb_ref[...]
    dh = _dot(h, dw_ref[...]) + db_ref[...]
    eh = _dot(h, ew_ref[...]) + eb_ref[...]
    dbt_out[0] = jnp.concatenate([dh[:, :HH], bh[:, :HH]], axis=1)
    dbt_out[1] = jnp.concatenate([dh[:, HH:], bh[:, HH:]], axis=1)
    et_out[...] = eh
    we0_out[...] = _dot(eew_ref[...], cw0_ref[...])
    be0_out[...] = _dot(eeb_ref[...], cw0_ref[...]) + cb0_ref[...]
    we1_out[...] = _dot(eew_ref[...], cw1_ref[...])
    be1_out[...] = _dot(eeb_ref[...], cw1_ref[...]) + cb1_ref[...]


def _tables_body(nd_ref, stats_ref, ah_ref, h_ref, bng_ref, bnb_ref,
                 beg_ref, beb_ref, aw_ref, ab_ref, bw_ref, bb_ref,
                 dw_ref, db_ref, ew_ref, eb_ref,
                 h_out, ah_out, dbt_out, et_out, eas_out, eab_out):
    num = jnp.concatenate([nd_ref[0, :N, :HH], nd_ref[1, :N, :HH]], axis=1)
    den = jnp.concatenate([nd_ref[0, :N, HH:], nd_ref[1, :N, HH:]], axis=1)
    hnew = ah_ref[...] + num / (den + 1e-6)
    m = jnp.mean(hnew, axis=0, keepdims=True)
    var = jnp.mean((hnew - m) * (hnew - m), axis=0, keepdims=True)
    hact = jax.nn.relu((hnew - m) / jnp.sqrt(var + 1e-5) * bng_ref[...]
                       + bnb_ref[...])
    h = h_ref[...] + hact
    h_out[...] = h
    ah_out[...] = _dot(h, aw_ref[...]) + ab_ref[...]
    bh = _dot(h, bw_ref[...]) + bb_ref[...]
    dh = _dot(h, dw_ref[...]) + db_ref[...]
    eh = _dot(h, ew_ref[...]) + eb_ref[...]
    dbt_out[0] = jnp.concatenate([dh[:, :HH], bh[:, :HH]], axis=1)
    dbt_out[1] = jnp.concatenate([dh[:, HH:], bh[:, HH:]], axis=1)
    et_out[...] = eh
    s = jnp.concatenate([jnp.sum(stats_ref[0, :, 0, :HH], axis=0),
                         jnp.sum(stats_ref[1, :, 0, :HH], axis=0)])
    sq = jnp.concatenate([jnp.sum(stats_ref[0, :, 0, HH:], axis=0),
                          jnp.sum(stats_ref[1, :, 0, HH:], axis=0)])
    em = s / E
    ev = sq / E - em * em
    a = beg_ref[0] / jnp.sqrt(ev + 1e-5)
    eas_out[...] = a.reshape(1, H)
    eab_out[...] = (beb_ref[0] - em * a).reshape(1, H)


def _ce0_body(e_ref, we_ref, be_ref, ce_out):
    t = _dot(e_ref[...], we_ref[...]) + be_ref[...]
    ce_out[0] = t[:, :HH]
    ce_out[1] = t[:, HH:]


def _ce1_body(e_ref, enew_ref, eas_ref, eab_ref, we_ref, cw_ref, be_ref,
              ce_out):
    ec = jnp.concatenate([enew_ref[0], enew_ref[1]], axis=1)
    eact = jax.nn.relu(ec * eas_ref[...] + eab_ref[...])
    t = _dot(e_ref[...], we_ref[...]) + _dot(eact, cw_ref[...]) + be_ref[...]
    ce_out[0] = t[:, :HH]
    ce_out[1] = t[:, HH:]


def _final_body(nd_ref, ah_ref, h_ref, bng_ref, bnb_ref, w1_ref, b1_ref,
                w2_ref, b2_ref, o_out):
    num = jnp.concatenate([nd_ref[0, :N, :HH], nd_ref[1, :N, :HH]], axis=1)
    den = jnp.concatenate([nd_ref[0, :N, HH:], nd_ref[1, :N, HH:]], axis=1)
    hnew = ah_ref[...] + num / (den + 1e-6)
    m = jnp.mean(hnew, axis=0, keepdims=True)
    var = jnp.mean((hnew - m) * (hnew - m), axis=0, keepdims=True)
    hact = jax.nn.relu((hnew - m) / jnp.sqrt(var + 1e-5) * bng_ref[...]
                       + bnb_ref[...])
    h = h_ref[...] + hact
    o = jax.nn.relu(_dot(h, w1_ref[...]) + b1_ref[...])
    o = _dot(o, w2_ref[...]) + b2_ref[...]
    o_out[...] = jnp.clip(o, -MAX_ACTION, MAX_ACTION)


def _sc_edge_body(with_enew, dbt_hbm, et_hbm, ce_hbm, srcr_hbm, dstr_hbm,
                  z_hbm, *rest):
    if with_enew:
        (nd_hbm, enew_hbm, stats_hbm, acc_sh,
         si0, si1, di0, di1, db0, db1, et0, et1, ce0, ce1, ps0, ps1,
         stat_acc, gsem0, gsem1, scsem0, scsem1, isem0, isem1,
         esem0, esem1) = rest
    else:
        (nd_hbm, acc_sh,
         si0, si1, di0, di1, db0, db1, et0, et1, ce0, ce1, ps0, ps1,
         gsem0, gsem1, scsem0, scsem1, isem0, isem1) = rest
        stat_acc = None
        esem0 = esem1 = None
        enew_hbm = None
    si = (si0, si1)
    di = (di0, di1)
    db_buf = (db0, db1)
    et_buf = (et0, et1)
    ce_buf = (ce0, ce1)
    ps_buf = (ps0, ps1)
    gsem = (gsem0, gsem1)
    scsem = (scsem0, scsem1)
    isem = (isem0, isem1)
    esem = (esem0, esem1)
    c = lax.axis_index("c")
    s = lax.axis_index("s")
    coff = c * HH

    pltpu.sync_copy(z_hbm.at[pl.ds(s * NPT, NPT)],
                    acc_sh.at[pl.ds(s * NPT, NPT)])
    if with_enew:
        z16 = jnp.zeros((16,), _f32)
        for c8 in range(8):
            stat_acc.at[0, pl.ds(c8 * 16, 16)][...] = z16
    plsc.subcore_barrier()

    def idx_start(b, i):
        cid = s * NCHUNK + i
        pltpu.async_copy(srcr_hbm.at[cid], si[b], isem[b])
        pltpu.async_copy(dstr_hbm.at[cid], di[b], isem[b])

    def gathers_start(b, i):
        base = s * PER_TILE + i * K
        pltpu.make_async_copy(srcr_hbm.at[0], si[b], isem[b]).wait()
        pltpu.make_async_copy(dstr_hbm.at[0], di[b], isem[b]).wait()
        pltpu.async_copy(dbt_hbm.at[c].at[si[b].at[0]], db_buf[b],
                         gsem[b])
        pltpu.async_copy(et_hbm.at[di[b].at[0]], et_buf[b], gsem[b])
        pltpu.async_copy(ce_hbm.at[c].at[pl.ds(base, K)], ce_buf[b], gsem[b])

    def wait_gathers(b):
        pltpu.make_async_copy(dbt_hbm.at[c].at[si[b].at[0]], db_buf[b],
                              gsem[b]).wait()
        pltpu.make_async_copy(et_hbm.at[di[b].at[0]], et_buf[b],
                              gsem[b]).wait()
        pltpu.make_async_copy(ce_hbm.at[c].at[pl.ds(0, K)], ce_buf[b],
                              gsem[b]).wait()

    def compute(b):
        def _one(r, carry):
            new = []
            for c2 in range(4):
                lo = (r, pl.ds(c2 * 16, 16))
                hi = (r, pl.ds(HH + c2 * 16, 16))
                le = (r, pl.ds(coff + c2 * 16, 16))
                en = (db_buf[b].at[*lo][...] + et_buf[b].at[*le][...]
                      + ce_buf[b].at[*lo][...])
                sg = _sigmoid(en)
                ps_buf[b].at[*lo][...] = sg * db_buf[b].at[*hi][...]
                ps_buf[b].at[*hi][...] = sg
                if with_enew:
                    ce_buf[b].at[*lo][...] = en
                    su, sq = carry[c2]
                    new.append((su + en, sq + en * en))
            return tuple(new)

        if with_enew:
            z = jnp.zeros((16,), _f32)
            carry0 = tuple((z, z) for _ in range(4))
            carry = plsc.parallel_loop(0, K, 1, unroll=8,
                                       carry=carry0)(_one)
            for c2 in range(4):
                su, sq = carry[c2]
                plsc.addupdate(stat_acc.at[0, pl.ds(c2 * 16, 16)], su)
                plsc.addupdate(
                    stat_acc.at[0, pl.ds(HH + c2 * 16, 16)], sq)
        else:
            @plsc.parallel_loop(0, K, 1, unroll=8)
            def _rows(r):
                _one(r, None)

    def issue_out(b, i):
        base = s * PER_TILE + i * K
        if with_enew:
            pltpu.async_copy(ce_buf[b], enew_hbm.at[c].at[pl.ds(base, K)],
                             esem[b])
        pltpu.async_copy(ps_buf[b], acc_sh.at[di[b].at[0]], scsem[b],
                         add=True)

    def drain_out(b):
        if with_enew:
            pltpu.make_async_copy(ce_buf[b],
                                  enew_hbm.at[c].at[pl.ds(0, K)],
                                  esem[b]).wait()
        pltpu.make_async_copy(ps_buf[b], acc_sh.at[di[b].at[0]],
                              scsem[b]).wait()

    NPAIR = NCHUNK // 2
    idx_start(0, jnp.int32(0))
    gathers_start(0, jnp.int32(0))

    @pl.loop(0, NPAIR)
    def _pair(p):
        i0 = 2 * p

        @pl.when(p > 0)
        def _():
            drain_out(1)

        idx_start(1, i0 + 1)
        wait_gathers(0)
        gathers_start(1, i0 + 1)
        compute(0)
        issue_out(0, i0)
        wait_gathers(1)
        compute(1)
        issue_out(1, i0 + 1)

        @pl.when(p < NPAIR - 1)
        def _():
            drain_out(0)
            idx_start(0, i0 + 2)
            gathers_start(0, i0 + 2)

    drain_out(0)
    drain_out(1)

    plsc.subcore_barrier()
    pltpu.sync_copy(acc_sh.at[pl.ds(s * NPT, NPT)],
                    nd_hbm.at[c].at[pl.ds(s * NPT, NPT)])
    if with_enew:
        pltpu.sync_copy(stat_acc, stats_hbm.at[c].at[s])


@functools.lru_cache(maxsize=None)
def _make_sc_edge(with_enew):
    out_type = [jax.ShapeDtypeStruct((NCORE, NPAD, H), _f32)]
    if with_enew:
        out_type = out_type + [
            jax.ShapeDtypeStruct((NCORE, E, HH), _f32),
            jax.ShapeDtypeStruct((NCORE, NSUB, 1, H), _f32),
        ]
    scratch = [pltpu.VMEM_SHARED((NPAD, H), _f32)]
    scratch += [pltpu.VMEM((1, K), jnp.int32)] * 4
    scratch += [pltpu.VMEM((K, H), _f32)] * 4
    scratch += [pltpu.VMEM((K, HH), _f32)] * 2
    scratch += [pltpu.VMEM((K, H), _f32)] * 2
    if with_enew:
        scratch += [pltpu.VMEM((1, H), _f32)]
    nsem = 8 if with_enew else 6
    scratch += [pltpu.SemaphoreType.DMA] * nsem
    mesh = plsc.VectorSubcoreMesh(core_axis_name="c", subcore_axis_name="s")
    import dataclasses
    cp = pltpu.CompilerParams()
    if "needs_layout_passes" in pltpu.CompilerParams.__dataclass_fields__:
        cp = dataclasses.replace(cp, needs_layout_passes=False)
    return pl.kernel(functools.partial(_sc_edge_body, with_enew),
                     out_type=out_type, mesh=mesh, scratch_types=scratch,
                     compiler_params=cp)


def kernel(x, e, emb_h_w, emb_h_b, emb_e_w, emb_e_b, A_w, A_b, B_w, B_b, C_w,
           C_b, D_w, D_b, E_w, E_b, bn_h_g, bn_h_b, bn_e_g, bn_e_b, out1_w,
           out1_b, out2_w, out2_b, edge_index):
    r1 = lambda v: v.reshape(1, -1)
    f32 = jnp.float32
    BE = 2000
    GE = E // BE

    h0, ah0, dbt0, et0, we0, be0, we1, be1 = pl.pallas_call(
        _prep_body,
        out_shape=[
            jax.ShapeDtypeStruct((N, H), f32),
            jax.ShapeDtypeStruct((N, H), f32),
            jax.ShapeDtypeStruct((NCORE, N, H), f32),
            jax.ShapeDtypeStruct((N, H), f32),
            jax.ShapeDtypeStruct((DE, H), f32),
            jax.ShapeDtypeStruct((1, H), f32),
            jax.ShapeDtypeStruct((DE, H), f32),
            jax.ShapeDtypeStruct((1, H), f32),
        ],
    )(x, emb_h_w, r1(emb_h_b), A_w[0], r1(A_b[0]), B_w[0], r1(B_b[0]),
      D_w[0], r1(D_b[0]), E_w[0], r1(E_b[0]), emb_e_w, r1(emb_e_b),
      C_w[0], r1(C_b[0]), C_w[1], r1(C_b[1]))

    ce0 = pl.pallas_call(
        _ce0_body,
        grid=(GE,),
        in_specs=[
            pl.BlockSpec((BE, DE), lambda i: (i, 0)),
            pl.BlockSpec((DE, H), lambda i: (0, 0)),
            pl.BlockSpec((1, H), lambda i: (0, 0)),
        ],
        out_specs=pl.BlockSpec((NCORE, BE, HH), lambda i: (0, i, 0)),
        out_shape=jax.ShapeDtypeStruct((NCORE, E, HH), f32),
    )(e, we0, be0)

    zeros = jnp.zeros((NPAD, H), f32)
    src_r = edge_index[0].reshape(E // K, 1, K)
    dst_r = edge_index[1].reshape(E // K, 1, K)
    nd0, enew0, stats0 = _make_sc_edge(True)(dbt0, et0, ce0, src_r, dst_r,
                                             zeros)

    h1, ah1, dbt1, et1, eas, eab = pl.pallas_call(
        _tables_body,
        out_shape=[
            jax.ShapeDtypeStruct((N, H), f32),
            jax.ShapeDtypeStruct((N, H), f32),
            jax.ShapeDtypeStruct((NCORE, N, H), f32),
            jax.ShapeDtypeStruct((N, H), f32),
            jax.ShapeDtypeStruct((1, H), f32),
            jax.ShapeDtypeStruct((1, H), f32),
        ],
    )(nd0, stats0, ah0, h0, r1(bn_h_g[0]), r1(bn_h_b[0]), r1(bn_e_g[0]),
      r1(bn_e_b[0]), A_w[1], r1(A_b[1]), B_w[1], r1(B_b[1]), D_w[1],
      r1(D_b[1]), E_w[1], r1(E_b[1]))

    ce1 = pl.pallas_call(
        _ce1_body,
        grid=(GE,),
        in_specs=[
            pl.BlockSpec((BE, DE), lambda i: (i, 0)),
            pl.BlockSpec((NCORE, BE, HH), lambda i: (0, i, 0)),
            pl.BlockSpec((1, H), lambda i: (0, 0)),
            pl.BlockSpec((1, H), lambda i: (0, 0)),
            pl.BlockSpec((DE, H), lambda i: (0, 0)),
            pl.BlockSpec((H, H), lambda i: (0, 0)),
            pl.BlockSpec((1, H), lambda i: (0, 0)),
        ],
        out_specs=pl.BlockSpec((NCORE, BE, HH), lambda i: (0, i, 0)),
        out_shape=jax.ShapeDtypeStruct((NCORE, E, HH), f32),
    )(e, enew0, eas, eab, we1, C_w[1], be1)

    nd1 = _make_sc_edge(False)(dbt1, et1, ce1, src_r, dst_r, zeros)
    if isinstance(nd1, (tuple, list)):
        nd1 = nd1[0]

    o = pl.pallas_call(
        _final_body,
        out_shape=jax.ShapeDtypeStruct((N, OUT), f32),
    )(nd1, ah1, h1, r1(bn_h_g[1]), r1(bn_h_b[1]), out1_w, r1(out1_b),
      out2_w, r1(out2_b))
    return o


# speedup vs baseline: 1.5151x; 1.5151x over previous
import functools

import jax
import jax.numpy as jnp
from jax import lax
from jax.experimental import pallas as pl
from jax.experimental.pallas import tpu as pltpu
from jax.experimental.pallas import tpu_sc as plsc

N = 10000
E = 160000
DIN = 128
DE = 16
H = 128
HH = 64
OUT = 8
MAX_ACTION = 1.0

NCORE = 2
NSUB = 16
PER_TILE = E // NSUB
K = 40
NCHUNK = PER_TILE // K
NPAD = 10112
NPT = NPAD // NSUB

_f32 = jnp.float32
_RCP_MAGIC = 0x7EF127EA


def _sigmoid(en):
    w = 1.0 + jnp.exp(-jnp.maximum(en, -60.0))
    y = plsc.bitcast(_RCP_MAGIC - plsc.bitcast(w, jnp.int32), _f32)
    y = y * (2.0 - w * y)
    y = y * (2.0 - w * y)
    return y


def _dot(a, b):
    return jax.lax.dot_general(a, b, (((1,), (0,)), ((), ())),
                               preferred_element_type=_f32)


def _prep_body(x_ref, ehw_ref, ehb_ref, aw_ref, ab_ref, bw_ref, bb_ref,
               dw_ref, db_ref, ew_ref, eb_ref, eew_ref, eeb_ref,
               cw0_ref, cb0_ref, cw1_ref, cb1_ref,
               h_out, ah_out, dbt_out, et_out, we0_out, be0_out,
               we1_out, be1_out):
    h = _dot(x_ref[...], ehw_ref[...]) + ehb_ref[...]
    h_out[...] = h
    ah_out[...] = _dot(h, aw_ref[...]) + ab_ref[...]
    bh = _dot(h, bw_ref[...]) + bb_ref[...]
    dh = _dot(h, dw_ref[...]) + db_ref[...]
    eh = _dot(h, ew_ref[...]) + eb_ref[...]
    dbt_out[0] = jnp.concatenate([dh[:, :HH], bh[:, :HH]], axis=1)
    dbt_out[1] = jnp.concatenate([dh[:, HH:], bh[:, HH:]], axis=1)
    et_out[...] = eh
    we0_out[...] = _dot(eew_ref[...], cw0_ref[...])
    be0_out[...] = _dot(eeb_ref[...], cw0_ref[...]) + cb0_ref[...]
    we1_out[...] = _dot(eew_ref[...], cw1_ref[...])
    be1_out[...] = _dot(eeb_ref[...], cw1_ref[...]) + cb1_ref[...]


def _tables_body(nd_ref, stats_ref, ah_ref, h_ref, bng_ref, bnb_ref,
                 beg_ref, beb_ref, aw_ref, ab_ref, bw_ref, bb_ref,
                 dw_ref, db_ref, ew_ref, eb_ref,
                 h_out, ah_out, dbt_out, et_out, eas_out, eab_out):
    num = jnp.concatenate([nd_ref[0, :N, :HH], nd_ref[1, :N, :HH]], axis=1)
    den = jnp.concatenate([nd_ref[0, :N, HH:], nd_ref[1, :N, HH:]], axis=1)
    hnew = ah_ref[...] + num / (den + 1e-6)
    m = jnp.mean(hnew, axis=0, keepdims=True)
    var = jnp.mean((hnew - m) * (hnew - m), axis=0, keepdims=True)
    hact = jax.nn.relu((hnew - m) / jnp.sqrt(var + 1e-5) * bng_ref[...]
                       + bnb_ref[...])
    h = h_ref[...] + hact
    h_out[...] = h
    ah_out[...] = _dot(h, aw_ref[...]) + ab_ref[...]
    bh = _dot(h, bw_ref[...]) + bb_ref[...]
    dh = _dot(h, dw_ref[...]) + db_ref[...]
    eh = _dot(h, ew_ref[...]) + eb_ref[...]
    dbt_out[0] = jnp.concatenate([dh[:, :HH], bh[:, :HH]], axis=1)
    dbt_out[1] = jnp.concatenate([dh[:, HH:], bh[:, HH:]], axis=1)
    et_out[...] = eh
    s = jnp.concatenate([jnp.sum(stats_ref[0, :, 0, :HH], axis=0),
                         jnp.sum(stats_ref[1, :, 0, :HH], axis=0)])
    sq = jnp.concatenate([jnp.sum(stats_ref[0, :, 0, HH:], axis=0),
                          jnp.sum(stats_ref[1, :, 0, HH:], axis=0)])
    em = s / E
    ev = sq / E - em * em
    a = beg_ref[0] / jnp.sqrt(ev + 1e-5)
    eas_out[...] = a.reshape(1, H)
    eab_out[...] = (beb_ref[0] - em * a).reshape(1, H)


def _ce0_body(e_ref, we_ref, be_ref, ce_out):
    t = _dot(e_ref[...], we_ref[...]) + be_ref[...]
    ce_out[0] = t[:, :HH]
    ce_out[1] = t[:, HH:]


def _ce1_body(e_ref, enew_ref, eas_ref, eab_ref, we_ref, cw_ref, be_ref,
              ce_out):
    ec = jnp.concatenate([enew_ref[0], enew_ref[1]], axis=1)
    eact = jax.nn.relu(ec * eas_ref[...] + eab_ref[...])
    t = _dot(e_ref[...], we_ref[...]) + _dot(eact, cw_ref[...]) + be_ref[...]
    ce_out[0] = t[:, :HH]
    ce_out[1] = t[:, HH:]


def _final_body(nd_ref, ah_ref, h_ref, bng_ref, bnb_ref, w1_ref, b1_ref,
                w2_ref, b2_ref, o_out):
    num = jnp.concatenate([nd_ref[0, :N, :HH], nd_ref[1, :N, :HH]], axis=1)
    den = jnp.concatenate([nd_ref[0, :N, HH:], nd_ref[1, :N, HH:]], axis=1)
    hnew = ah_ref[...] + num / (den + 1e-6)
    m = jnp.mean(hnew, axis=0, keepdims=True)
    var = jnp.mean((hnew - m) * (hnew - m), axis=0, keepdims=True)
    hact = jax.nn.relu((hnew - m) / jnp.sqrt(var + 1e-5) * bng_ref[...]
                       + bnb_ref[...])
    h = h_ref[...] + hact
    o = jax.nn.relu(_dot(h, w1_ref[...]) + b1_ref[...])
    o = _dot(o, w2_ref[...]) + b2_ref[...]
    o_out[...] = jnp.clip(o, -MAX_ACTION, MAX_ACTION)


def _sc_edge_body(with_enew, dbt_hbm, et_hbm, ce_hbm, srcr_hbm, dstr_hbm,
                  z_hbm, *rest):
    if with_enew:
        (nd_hbm, enew_hbm, stats_hbm, acc_sh,
         si0, si1, di0, di1, db0, db1, et0, et1, ce0, ce1, ps0, ps1,
         stat_acc, gsem0, gsem1, scsem0, scsem1, isem0, isem1,
         esem0, esem1) = rest
    else:
        (nd_hbm, acc_sh,
         si0, si1, di0, di1, db0, db1, et0, et1, ce0, ce1, ps0, ps1,
         gsem0, gsem1, scsem0, scsem1, isem0, isem1) = rest
        stat_acc = None
        esem0 = esem1 = None
        enew_hbm = None
    si = (si0, si1)
    di = (di0, di1)
    db_buf = (db0, db1)
    et_buf = (et0, et1)
    ce_buf = (ce0, ce1)
    ps_buf = (ps0, ps1)
    gsem = (gsem0, gsem1)
    scsem = (scsem0, scsem1)
    isem = (isem0, isem1)
    esem = (esem0, esem1)
    c = lax.axis_index("c")
    s = lax.axis_index("s")
    coff = c * HH

    pltpu.sync_copy(z_hbm.at[pl.ds(s * NPT, NPT)],
                    acc_sh.at[pl.ds(s * NPT, NPT)])
    if with_enew:
        z16 = jnp.zeros((16,), _f32)
        for c8 in range(8):
            stat_acc.at[0, pl.ds(c8 * 16, 16)][...] = z16
    plsc.subcore_barrier()

    def idx_start(b, i):
        cid = s * NCHUNK + i
        pltpu.async_copy(srcr_hbm.at[cid], si[b], isem[b])
        pltpu.async_copy(dstr_hbm.at[cid], di[b], isem[b])

    def gathers_start(b, i):
        base = s * PER_TILE + i * K
        pltpu.make_async_copy(srcr_hbm.at[0], si[b], isem[b]).wait()
        pltpu.make_async_copy(dstr_hbm.at[0], di[b], isem[b]).wait()
        pltpu.async_copy(dbt_hbm.at[c].at[si[b].at[0]], db_buf[b],
                         gsem[b])
        pltpu.async_copy(et_hbm.at[di[b].at[0]], et_buf[b], gsem[b])
        pltpu.async_copy(ce_hbm.at[c].at[pl.ds(base, K)], ce_buf[b], gsem[b])

    def wait_gathers(b):
        pltpu.make_async_copy(dbt_hbm.at[c].at[si[b].at[0]], db_buf[b],
                              gsem[b]).wait()
        pltpu.make_async_copy(et_hbm.at[di[b].at[0]], et_buf[b],
                              gsem[b]).wait()
        pltpu.make_async_copy(ce_hbm.at[c].at[pl.ds(0, K)], ce_buf[b],
                              gsem[b]).wait()

    def compute(b):
        def _one(r, carry):
            new = []
            for c2 in range(4):
                lo = (r, pl.ds(c2 * 16, 16))
                hi = (r, pl.ds(HH + c2 * 16, 16))
                le = (r, pl.ds(coff + c2 * 16, 16))
                en = (db_buf[b].at[*lo][...] + et_buf[b].at[*le][...]
                      + ce_buf[b].at[*lo][...])
                sg = _sigmoid(en)
                ps_buf[b].at[*lo][...] = sg * db_buf[b].at[*hi][...]
                ps_buf[b].at[*hi][...] = sg
                if with_enew:
                    ce_buf[b].at[*lo][...] = en
                    su, sq = carry[c2]
                    new.append((su + en, sq + en * en))
            return tuple(new)

        if with_enew:
            z = jnp.zeros((16,), _f32)
            carry0 = tuple((z, z) for _ in range(4))
            carry = plsc.parallel_loop(0, K, 1, unroll=4,
                                       carry=carry0)(_one)
            for c2 in range(4):
                su, sq = carry[c2]
                plsc.addupdate(stat_acc.at[0, pl.ds(c2 * 16, 16)], su)
                plsc.addupdate(
                    stat_acc.at[0, pl.ds(HH + c2 * 16, 16)], sq)
        else:
            @plsc.parallel_loop(0, K, 1, unroll=4)
            def _rows(r):
                _one(r, None)

    def issue_out(b, i):
        base = s * PER_TILE + i * K
        if with_enew:
            pltpu.async_copy(ce_buf[b], enew_hbm.at[c].at[pl.ds(base, K)],
                             esem[b])
        pltpu.async_copy(ps_buf[b], acc_sh.at[di[b].at[0]], scsem[b],
                         add=True)

    def drain_out(b):
        if with_enew:
            pltpu.make_async_copy(ce_buf[b],
                                  enew_hbm.at[c].at[pl.ds(0, K)],
                                  esem[b]).wait()
        pltpu.make_async_copy(ps_buf[b], acc_sh.at[di[b].at[0]],
                              scsem[b]).wait()

    NPAIR = NCHUNK // 2
    idx_start(0, jnp.int32(0))
    gathers_start(0, jnp.int32(0))

    @pl.loop(0, NPAIR)
    def _pair(p):
        i0 = 2 * p

        @pl.when(p > 0)
        def _():
            drain_out(1)

        idx_start(1, i0 + 1)
        wait_gathers(0)
        gathers_start(1, i0 + 1)
        compute(0)
        issue_out(0, i0)
        wait_gathers(1)

        @pl.when(p < NPAIR - 1)
        def _():
            drain_out(0)
            idx_start(0, i0 + 2)
            gathers_start(0, i0 + 2)

        compute(1)
        issue_out(1, i0 + 1)

    drain_out(0)
    drain_out(1)

    plsc.subcore_barrier()
    pltpu.sync_copy(acc_sh.at[pl.ds(s * NPT, NPT)],
                    nd_hbm.at[c].at[pl.ds(s * NPT, NPT)])
    if with_enew:
        pltpu.sync_copy(stat_acc, stats_hbm.at[c].at[s])


@functools.lru_cache(maxsize=None)
def _make_sc_edge(with_enew):
    out_type = [jax.ShapeDtypeStruct((NCORE, NPAD, H), _f32)]
    if with_enew:
        out_type = out_type + [
            jax.ShapeDtypeStruct((NCORE, E, HH), _f32),
            jax.ShapeDtypeStruct((NCORE, NSUB, 1, H), _f32),
        ]
    scratch = [pltpu.VMEM_SHARED((NPAD, H), _f32)]
    scratch += [pltpu.VMEM((1, K), jnp.int32)] * 4
    scratch += [pltpu.VMEM((K, H), _f32)] * 4
    scratch += [pltpu.VMEM((K, HH), _f32)] * 2
    scratch += [pltpu.VMEM((K, H), _f32)] * 2
    if with_enew:
        scratch += [pltpu.VMEM((1, H), _f32)]
    nsem = 8 if with_enew else 6
    scratch += [pltpu.SemaphoreType.DMA] * nsem
    mesh = plsc.VectorSubcoreMesh(core_axis_name="c", subcore_axis_name="s")
    import dataclasses
    cp = pltpu.CompilerParams()
    if "needs_layout_passes" in pltpu.CompilerParams.__dataclass_fields__:
        cp = dataclasses.replace(cp, needs_layout_passes=False)
    return pl.kernel(functools.partial(_sc_edge_body, with_enew),
                     out_type=out_type, mesh=mesh, scratch_types=scratch,
                     compiler_params=cp)


def kernel(x, e, emb_h_w, emb_h_b, emb_e_w, emb_e_b, A_w, A_b, B_w, B_b, C_w,
           C_b, D_w, D_b, E_w, E_b, bn_h_g, bn_h_b, bn_e_g, bn_e_b, out1_w,
           out1_b, out2_w, out2_b, edge_index):
    r1 = lambda v: v.reshape(1, -1)
    f32 = jnp.float32
    BE = 2000
    GE = E // BE

    h0, ah0, dbt0, et0, we0, be0, we1, be1 = pl.pallas_call(
        _prep_body,
        out_shape=[
            jax.ShapeDtypeStruct((N, H), f32),
            jax.ShapeDtypeStruct((N, H), f32),
            jax.ShapeDtypeStruct((NCORE, N, H), f32),
            jax.ShapeDtypeStruct((N, H), f32),
            jax.ShapeDtypeStruct((DE, H), f32),
            jax.ShapeDtypeStruct((1, H), f32),
            jax.ShapeDtypeStruct((DE, H), f32),
            jax.ShapeDtypeStruct((1, H), f32),
        ],
    )(x, emb_h_w, r1(emb_h_b), A_w[0], r1(A_b[0]), B_w[0], r1(B_b[0]),
      D_w[0], r1(D_b[0]), E_w[0], r1(E_b[0]), emb_e_w, r1(emb_e_b),
      C_w[0], r1(C_b[0]), C_w[1], r1(C_b[1]))

    ce0 = pl.pallas_call(
        _ce0_body,
        grid=(GE,),
        in_specs=[
            pl.BlockSpec((BE, DE), lambda i: (i, 0)),
            pl.BlockSpec((DE, H), lambda i: (0, 0)),
            pl.BlockSpec((1, H), lambda i: (0, 0)),
        ],
        out_specs=pl.BlockSpec((NCORE, BE, HH), lambda i: (0, i, 0)),
        out_shape=jax.ShapeDtypeStruct((NCORE, E, HH), f32),
    )(e, we0, be0)

    zeros = jnp.zeros((NPAD, H), f32)
    src_r = edge_index[0].reshape(E // K, 1, K)
    dst_r = edge_index[1].reshape(E // K, 1, K)
    nd0, enew0, stats0 = _make_sc_edge(True)(dbt0, et0, ce0, src_r, dst_r,
                                             zeros)

    h1, ah1, dbt1, et1, eas, eab = pl.pallas_call(
        _tables_body,
        out_shape=[
            jax.ShapeDtypeStruct((N, H), f32),
            jax.ShapeDtypeStruct((N, H), f32),
            jax.ShapeDtypeStruct((NCORE, N, H), f32),
            jax.ShapeDtypeStruct((N, H), f32),
            jax.ShapeDtypeStruct((1, H), f32),
            jax.ShapeDtypeStruct((1, H), f32),
        ],
    )(nd0, stats0, ah0, h0, r1(bn_h_g[0]), r1(bn_h_b[0]), r1(bn_e_g[0]),
      r1(bn_e_b[0]), A_w[1], r1(A_b[1]), B_w[1], r1(B_b[1]), D_w[1],
      r1(D_b[1]), E_w[1], r1(E_b[1]))

    ce1 = pl.pallas_call(
        _ce1_body,
        grid=(GE,),
        in_specs=[
            pl.BlockSpec((BE, DE), lambda i: (i, 0)),
            pl.BlockSpec((NCORE, BE, HH), lambda i: (0, i, 0)),
            pl.BlockSpec((1, H), lambda i: (0, 0)),
            pl.BlockSpec((1, H), lambda i: (0, 0)),
            pl.BlockSpec((DE, H), lambda i: (0, 0)),
            pl.BlockSpec((H, H), lambda i: (0, 0)),
            pl.BlockSpec((1, H), lambda i: (0, 0)),
        ],
        out_specs=pl.BlockSpec((NCORE, BE, HH), lambda i: (0, i, 0)),
        out_shape=jax.ShapeDtypeStruct((NCORE, E, HH), f32),
    )(e, enew0, eas, eab, we1, C_w[1], be1)

    nd1 = _make_sc_edge(False)(dbt1, et1, ce1, src_r, dst_r, zeros)
    if isinstance(nd1, (tuple, list)):
        nd1 = nd1[0]

    o = pl.pallas_call(
        _final_body,
        out_shape=jax.ShapeDtypeStruct((N, OUT), f32),
    )(nd1, ah1, h1, r1(bn_h_g[1]), r1(bn_h_b[1]), out1_w, r1(out1_b),
      out2_w, r1(out2_b))
    return o
